# Initial kernel scaffold; baseline (speedup 1.0000x reference)
import jax
import jax.numpy as jnp
from jax.experimental import pallas as pl


def kernel(x, edge_rel0, edge_rel1, edge_self, W0, b0, W1, b1, Ws, bs):
    raise NotImplementedError("write your pallas kernel here")


# speedup vs baseline: 2.2206x; 2.2206x over previous
import functools

import jax
import jax.numpy as jnp
from jax import lax
from jax.experimental import pallas as pl
from jax.experimental.pallas import tpu as pltpu
from jax.experimental.pallas import tpu_sc as plsc

NC = 2
NS = 16
HD = 128
IR = 128
CR = 128
BM = 1000


def _sc_segment_sums(table, srcs, dsts, csrcs, cdsts, zacc, nr):
    nt = srcs.shape[3]
    rpt = nr // NS
    mesh = plsc.VectorSubcoreMesh(core_axis_name="c", subcore_axis_name="s")

    @functools.partial(
        pl.kernel,
        mesh=mesh,
        out_type=jax.ShapeDtypeStruct((NC, 2, nr, HD), jnp.float32),
        scratch_types=[
            pltpu.VMEM((8, IR), jnp.int32),
            pltpu.VMEM((8, IR), jnp.int32),
            pltpu.VMEM((IR, HD), jnp.float32),
            pltpu.VMEM_SHARED((nr, HD), jnp.float32),
            pltpu.SemaphoreType.DMA,
        ],
    )
    def sc_kernel(table_hbm, srcs_hbm, dsts_hbm, csrcs_hbm, cdsts_hbm,
                  zacc_hbm, a_hbm, src_v, dst_v, buf, acc, gsem):
        c = lax.axis_index("c")
        s = lax.axis_index("s")
        row0 = s * rpt

        for rel in range(2):
            pltpu.sync_copy(zacc_hbm, acc.at[pl.ds(row0, rpt)])
            plsc.subcore_barrier()

            for t in range(nt):
                pltpu.sync_copy(srcs_hbm.at[c, rel, s, t], src_v)
                pltpu.sync_copy(dsts_hbm.at[rel, s, t], dst_v)

                @pl.loop(0, 8)
                def _(j):
                    pltpu.async_copy(table_hbm.at[src_v.at[j]], buf,
                                     gsem).wait()
                    pltpu.sync_copy(buf, acc.at[dst_v.at[j]], add=True)

            @pl.when(c == rel)
            def _():
                for t in range(nt):
                    pltpu.sync_copy(csrcs_hbm.at[rel, s, t], src_v)
                    pltpu.sync_copy(cdsts_hbm.at[rel, s, t], dst_v)

                    @pl.loop(0, 8)
                    def _(j):
                        pltpu.async_copy(table_hbm.at[src_v.at[j]], buf,
                                         gsem).wait()
                        pltpu.sync_copy(buf, acc.at[dst_v.at[j]], add=True)

            plsc.subcore_barrier()
            pltpu.sync_copy(acc.at[pl.ds(row0, rpt)],
                            a_hbm.at[c, rel, pl.ds(row0, rpt)])
            plsc.subcore_barrier()

    return sc_kernel(table, srcs, dsts, csrcs, cdsts, zacc)


def _tc_body(x_ref, a_ref, cnt_ref, w_ref, b_ref, ws_ref, bs_ref, o_ref):
    f32 = jnp.float32
    out = jnp.dot(x_ref[...], ws_ref[...], preferred_element_type=f32)
    out += bs_ref[...][None, :]
    for rel in range(2):
        for c in range(NC):
            out += jnp.dot(a_ref[c, rel],
                           w_ref[rel][c * HD:(c + 1) * HD, :],
                           preferred_element_type=f32)
        out += cnt_ref[rel, 0, 0][:, None] * b_ref[rel][None, :]
    o_ref[...] = out


def _tc_combine(x, a, cnt, w, b, Ws, bs):
    n, d = x.shape
    dout = Ws.shape[1]
    bm = BM
    return pl.pallas_call(
        _tc_body,
        grid=(n // bm,),
        in_specs=[
            pl.BlockSpec((bm, d), lambda i: (i, 0)),
            pl.BlockSpec((NC, 2, bm, HD), lambda i: (0, 0, i, 0)),
            pl.BlockSpec((2, 1, 1, bm), lambda i: (0, i, 0, 0)),
            pl.BlockSpec((2, d, dout), lambda i: (0, 0, 0)),
            pl.BlockSpec((2, dout), lambda i: (0, 0)),
            pl.BlockSpec((d, dout), lambda i: (0, 0)),
            pl.BlockSpec((dout,), lambda i: (0,)),
        ],
        out_specs=pl.BlockSpec((bm, dout), lambda i: (i, 0)),
        out_shape=jax.ShapeDtypeStruct((n, dout), jnp.float32),
    )(x, a, cnt, w, b, Ws, bs)


def kernel(x, edge_rel0, edge_rel1, edge_self, W0, b0, W1, b1, Ws, bs):
    n, d = x.shape
    e = edge_rel0.shape[1]
    xh = x.reshape(n, 2, HD).transpose(1, 0, 2).reshape(2 * n, HD)
    table = jnp.concatenate([xh, jnp.eye(CR, dtype=jnp.float32)], axis=0)

    n_pad = ((n // NS + 7) // 8 * 8) * NS
    if n_pad == n:
        n_pad += 8 * NS
    nr = n_pad + CR
    ept = e // NS
    blk = 8 * IR
    ept_pad = (ept + blk - 1) // blk * blk
    nt = ept_pad // blk

    def prep(edge):
        src = edge[0].reshape(NS, ept)
        dst = edge[1].reshape(NS, ept)
        pad = ept_pad - ept
        srcp = jnp.pad(src, ((0, 0), (0, pad)))
        dstp = jnp.pad(dst, ((0, 0), (0, pad)), constant_values=n_pad - 1)
        csrc = jnp.pad(2 * n + dst % CR, ((0, 0), (0, pad)),
                       constant_values=2 * n)
        cdst = jnp.pad(n_pad + dst // CR, ((0, 0), (0, pad)),
                       constant_values=n_pad - 1)
        shp = (NS, nt, 8, IR)
        return (srcp.reshape(shp), dstp.reshape(shp),
                csrc.reshape(shp), cdst.reshape(shp))

    s0, d0, cs0, cd0 = prep(edge_rel0)
    s1, d1, cs1, cd1 = prep(edge_rel1)
    sbase = jnp.stack([s0, s1])
    srcs = jnp.stack([sbase, sbase + n])
    dsts = jnp.stack([d0, d1])
    csrcs = jnp.stack([cs0, cs1])
    cdsts = jnp.stack([cd0, cd1])
    zacc = jnp.zeros((nr // NS, HD), jnp.float32)

    a = _sc_segment_sums(table, srcs, dsts, csrcs, cdsts, zacc, nr)

    cnt = jnp.stack([a[0, 0, n_pad:].reshape(-1)[:n],
                     a[1, 1, n_pad:].reshape(-1)[:n]])
    cnt = cnt.reshape(2, n // BM, 1, BM)
    w = jnp.stack([W0, W1])
    b = jnp.stack([b0, b1])
    return _tc_combine(x, a, cnt, w, b, Ws, bs)
